# Initial kernel scaffold; baseline (speedup 1.0000x reference)
import jax
import jax.numpy as jnp
from jax.experimental import pallas as pl


def kernel(token_ids, embedding):
    raise NotImplementedError("write your pallas kernel here")


# speedup vs baseline: 1.7954x; 1.7954x over previous
import functools

import jax
import jax.numpy as jnp
from jax import lax
from jax.experimental import pallas as pl
from jax.experimental.pallas import tpu as pltpu
from jax.experimental.pallas import tpu_sc as plsc

EMBED_DIM = 64
NUM_CORES = 2
NUM_SUBCORES = 16
NUM_WORKERS = NUM_CORES * NUM_SUBCORES


@functools.partial(jax.jit, static_argnums=(2,))
def _gather_flat(ids, table, chunk):
    B = ids.shape[0]
    b_per_w = B // NUM_WORKERS
    n_chunks = b_per_w // chunk
    mesh = plsc.VectorSubcoreMesh(core_axis_name="c", subcore_axis_name="s")

    @functools.partial(
        pl.kernel,
        mesh=mesh,
        out_type=jax.ShapeDtypeStruct((B, EMBED_DIM), jnp.float32),
        scratch_types=[
            pltpu.VMEM((chunk,), jnp.int32),
            pltpu.VMEM((chunk, EMBED_DIM), jnp.float32),
            pltpu.SemaphoreType.DMA,
        ],
        compiler_params=pltpu.CompilerParams(use_tc_tiling_on_sc=False),
    )
    def k(ids_hbm, table_hbm, out_hbm, idx_v, rows_v, sem):
        wid = lax.axis_index("s") * NUM_CORES + lax.axis_index("c")
        base = wid * b_per_w

        def body(g, carry):
            start = pl.multiple_of(base + g * chunk, 8)
            pltpu.sync_copy(ids_hbm.at[pl.ds(start, chunk)], idx_v)
            pltpu.async_copy(table_hbm.at[idx_v], rows_v, sem).wait()
            pltpu.sync_copy(rows_v, out_hbm.at[pl.ds(start, chunk)])
            return carry

        lax.fori_loop(0, n_chunks, body, 0)

    return k(ids, table)


def kernel(token_ids, embedding):
    S, T = token_ids.shape
    ids = token_ids.reshape(S * T).astype(jnp.int32)
    out = _gather_flat(ids, embedding, 512)
    return out.reshape(S, T, EMBED_DIM)


# speedup vs baseline: 1.8611x; 1.0366x over previous
import functools

import jax
import jax.numpy as jnp
from jax import lax
from jax.experimental import pallas as pl
from jax.experimental.pallas import tpu as pltpu
from jax.experimental.pallas import tpu_sc as plsc

EMBED_DIM = 64
NUM_CORES = 2
NUM_SUBCORES = 16
NUM_WORKERS = NUM_CORES * NUM_SUBCORES


@functools.partial(jax.jit, static_argnums=(2,))
def _gather_flat(ids, table, chunk):
    B = ids.shape[0]
    b_per_w = B // NUM_WORKERS
    n_chunks = b_per_w // chunk
    assert n_chunks >= 2 and n_chunks % 2 == 0
    mesh = plsc.VectorSubcoreMesh(core_axis_name="c", subcore_axis_name="s")

    @functools.partial(
        pl.kernel,
        mesh=mesh,
        out_type=jax.ShapeDtypeStruct((B, EMBED_DIM), jnp.float32),
        scratch_types=[
            pltpu.VMEM((chunk,), jnp.int32),
            pltpu.VMEM((chunk,), jnp.int32),
            pltpu.VMEM((chunk, EMBED_DIM), jnp.float32),
            pltpu.VMEM((chunk, EMBED_DIM), jnp.float32),
            pltpu.SemaphoreType.DMA,
            pltpu.SemaphoreType.DMA,
            pltpu.SemaphoreType.DMA,
            pltpu.SemaphoreType.DMA,
        ],
        compiler_params=pltpu.CompilerParams(use_tc_tiling_on_sc=False),
    )
    def k(ids_hbm, table_hbm, out_hbm, idx0, idx1, rows0, rows1,
          gsem0, gsem1, ssem0, ssem1):
        wid = lax.axis_index("s") * NUM_CORES + lax.axis_index("c")
        base = wid * b_per_w
        idx = (idx0, idx1)
        rows = (rows0, rows1)
        gsem = (gsem0, gsem1)
        ssem = (ssem0, ssem1)

        def sl(g):
            return pl.ds(pl.multiple_of(base + g * chunk, 8), chunk)

        def stage(g, b):
            @pl.when(g >= 2)
            def _():
                pltpu.make_async_copy(rows[b], out_hbm.at[sl(g)], ssem[b]).wait()

            pltpu.sync_copy(ids_hbm.at[sl(g)], idx[b])
            pltpu.async_copy(table_hbm.at[idx[b]], rows[b], gsem[b])

            @pl.when(g >= 1)
            def _():
                o = 1 - b
                pltpu.make_async_copy(
                    table_hbm.at[idx[o]], rows[o], gsem[o]
                ).wait()
                pltpu.async_copy(rows[o], out_hbm.at[sl(g - 1)], ssem[o])

        def body(i, carry):
            stage(2 * i, 0)
            stage(2 * i + 1, 1)
            return carry

        lax.fori_loop(0, n_chunks // 2, body, 0)

        gl = n_chunks - 1
        pltpu.make_async_copy(table_hbm.at[idx[1]], rows[1], gsem[1]).wait()
        pltpu.async_copy(rows[1], out_hbm.at[sl(gl)], ssem[1])
        pltpu.make_async_copy(rows[0], out_hbm.at[sl(gl - 1)], ssem[0]).wait()
        pltpu.make_async_copy(rows[1], out_hbm.at[sl(gl)], ssem[1]).wait()

    return k(ids, table)


def kernel(token_ids, embedding):
    S, T = token_ids.shape
    ids = token_ids.reshape(S * T).astype(jnp.int32)
    out = _gather_flat(ids, embedding, 800)
    return out.reshape(S, T, EMBED_DIM)
